# Initial kernel scaffold; baseline (speedup 1.0000x reference)
import jax
import jax.numpy as jnp
from jax.experimental import pallas as pl


def kernel(input, word_embed_weight):
    raise NotImplementedError("write your pallas kernel here")


# speedup vs baseline: 6.2472x; 6.2472x over previous
import functools

import jax
import jax.numpy as jnp
from jax import lax
from jax.experimental import pallas as pl
from jax.experimental.pallas import tpu as pltpu
from jax.experimental.pallas import tpu_sc as plsc

_NTOKEN = 100000
_NINP = 64
_BATCH = 16384
_HIST = 50

_B = _BATCH * _HIST
_NC = 2
_NS = 16
_NW = _NC * _NS
_BPW = _B // _NW
_C = 128
_NCHUNK = _BPW // _C
_NBUF = 8
_NOUT = _NCHUNK // _NBUF


def _make_gather():
    mesh = plsc.VectorSubcoreMesh(core_axis_name="c", subcore_axis_name="s")

    @functools.partial(
        pl.kernel,
        mesh=mesh,
        out_type=jax.ShapeDtypeStruct((_B, _NINP), jnp.float32),
        scratch_types=[
            pltpu.VMEM((_NCHUNK, _C), jnp.int32),
            pltpu.VMEM((_NBUF, _C, _NINP), jnp.float32),
            pltpu.SemaphoreType.DMA,
            pltpu.SemaphoreType.DMA,
        ],
        compiler_params=pltpu.CompilerParams(use_tc_tiling_on_sc=False),
    )
    def gather(table_hbm, idx_hbm, out_hbm, idx_v, rows_v, gsem, wsem):
        wid = lax.axis_index("s") * _NC + lax.axis_index("c")
        base = wid * _BPW
        pltpu.sync_copy(idx_hbm.at[wid], idx_v)

        def body(it, carry):
            j0 = it * _NBUF
            gets = []
            for b in range(_NBUF):
                gets.append(
                    pltpu.async_copy(
                        table_hbm.at[idx_v.at[j0 + b]], rows_v.at[b], gsem
                    )
                )
            puts = []
            for b in range(_NBUF):
                gets[b].wait()
                puts.append(
                    pltpu.async_copy(
                        rows_v.at[b],
                        out_hbm.at[pl.ds(base + (j0 + b) * _C, _C)],
                        wsem,
                    )
                )
            for b in range(_NBUF):
                puts[b].wait()
            return carry

        lax.fori_loop(0, _NOUT, body, 0)

    return gather


_gather = _make_gather()


def kernel(input, word_embed_weight):
    idx = input.reshape(_NW, _NCHUNK, _C)
    out = _gather(word_embed_weight, idx)
    return out.reshape(_BATCH, _HIST, _NINP)
